# Initial kernel scaffold; baseline (speedup 1.0000x reference)
import jax
import jax.numpy as jnp
from jax.experimental import pallas as pl


def kernel(pred, target):
    raise NotImplementedError("write your pallas kernel here")


# speedup vs baseline: 1.8377x; 1.8377x over previous
import functools

import jax
import jax.numpy as jnp
from jax.experimental import pallas as pl
from jax.experimental.pallas import tpu as pltpu

N = 2048
CHUNK = 256
INF = float("inf")


def _chamfer_body(px_ref, py_ref, tx_ref, ty_ref, p2t_ref, t2p_ref):
    b = pl.program_id(0)
    tx = tx_ref[0]
    ty = ty_ref[0]
    txb = tx.astype(jnp.bfloat16).astype(jnp.float32)
    tyb = ty.astype(jnp.bfloat16).astype(jnp.float32)
    b2 = tx * tx + ty * ty

    def body(i, carry):
        rowsum, colkey, colval = carry
        pxc = px_ref[0, pl.ds(i * CHUNK, CHUNK), :]
        pyc = py_ref[0, pl.ds(i * CHUNK, CHUNK), :]
        pxb = pxc.astype(jnp.bfloat16).astype(jnp.float32)
        pyb = pyc.astype(jnp.bfloat16).astype(jnp.float32)
        a2 = pxc * pxc + pyc * pyc

        ab = pxb * txb + pyb * tyb
        key = (a2 + b2) - 2.0 * ab
        dx = pxc - tx
        dy = pyc - ty
        exact = dx * dx + dy * dy

        rowkey = jnp.min(key, axis=1, keepdims=True)
        rowval = jnp.min(jnp.where(key == rowkey, exact, INF),
                         axis=1, keepdims=True)
        rowsum = rowsum + jnp.sum(rowval, keepdims=True)

        ckey = jnp.min(key, axis=0, keepdims=True)
        cval = jnp.min(jnp.where(key == ckey, exact, INF),
                       axis=0, keepdims=True)
        take = ckey < colkey
        colkey = jnp.where(take, ckey, colkey)
        colval = jnp.where(take, cval, colval)
        return rowsum, colkey, colval

    init = (jnp.zeros((1, 1), jnp.float32),
            jnp.full((1, N), INF, dtype=jnp.float32),
            jnp.full((1, N), INF, dtype=jnp.float32))
    rowsum, _, colval = jax.lax.fori_loop(0, N // CHUNK, body, init)

    @pl.when(b == 0)
    def _():
        p2t_ref[:, :] = jnp.zeros((1, 1), jnp.float32)
        t2p_ref[:, :] = jnp.zeros((1, 1), jnp.float32)

    p2t_ref[:, :] += rowsum
    t2p_ref[:, :] += jnp.sum(colval, keepdims=True)


@functools.partial(jax.jit, static_argnames=("interpret",))
def kernel(pred, target, interpret=False):
    pred = pred.reshape(-1, N, 2)
    target = target.reshape(-1, N, 2)
    batch = pred.shape[0]

    pxT = pred[:, :, 0:1]
    pyT = pred[:, :, 1:2]
    tx = target[:, :, 0].reshape(batch, 1, N)
    ty = target[:, :, 1].reshape(batch, 1, N)

    p2t, t2p = pl.pallas_call(
        _chamfer_body,
        grid=(batch,),
        in_specs=[
            pl.BlockSpec((1, N, 1), lambda b: (b, 0, 0)),
            pl.BlockSpec((1, N, 1), lambda b: (b, 0, 0)),
            pl.BlockSpec((1, 1, N), lambda b: (b, 0, 0)),
            pl.BlockSpec((1, 1, N), lambda b: (b, 0, 0)),
        ],
        out_specs=[
            pl.BlockSpec((1, 1), lambda b: (0, 0)),
            pl.BlockSpec((1, 1), lambda b: (0, 0)),
        ],
        out_shape=[
            jax.ShapeDtypeStruct((1, 1), jnp.float32),
            jax.ShapeDtypeStruct((1, 1), jnp.float32),
        ],
        interpret=interpret,
    )(pxT, pyT, tx, ty)

    denom = jnp.float32(N * batch)
    return (p2t[0, 0] / denom, t2p[0, 0] / denom)
